# baseline (speedup 1.0000x reference)
import jax
import jax.numpy as jnp
from jax.experimental import pallas as pl
from jax.experimental.pallas import tpu as pltpu

NE = 8
DM = 768
DF = 768
TM = 2048


def _moe_body(x_ref, gw_ref, gu_ref, dn_ref, out_ref, sc_ref):
    t = pl.program_id(0)
    e = pl.program_id(1)

    @pl.when(e == 0)
    def _router():
        x = x_ref[...]
        logits = jnp.dot(x, gw_ref[...], preferred_element_type=jnp.float32)
        iota = jax.lax.broadcasted_iota(jnp.int32, logits.shape, 1)
        m1 = jnp.max(logits, axis=1, keepdims=True)
        a1 = jnp.min(jnp.where(logits == m1, iota, NE), axis=1, keepdims=True)
        masked = jnp.where(iota == a1, -jnp.inf, logits)
        m2 = jnp.max(masked, axis=1, keepdims=True)
        a2 = jnp.min(jnp.where(masked == m2, iota, NE), axis=1, keepdims=True)
        w0 = 1.0 / (1.0 + jnp.exp(m2 - m1))
        w1 = 1.0 - w0
        sc_ref[...] = (jnp.where(iota == a1, w0, 0.0)
                       + jnp.where(iota == a2, w1, 0.0))
        out_ref[...] = jnp.zeros_like(out_ref)

    x = x_ref[...]
    gu = jnp.dot(x, gu_ref[0], preferred_element_type=jnp.float32)
    g = gu[:, :DF]
    u = gu[:, DF:]
    h = u * (g * jax.nn.sigmoid(g))
    y = jnp.dot(h, dn_ref[0], preferred_element_type=jnp.float32)
    eiota = jax.lax.broadcasted_iota(jnp.int32, (TM, NE), 1)
    w = jnp.sum(jnp.where(eiota == e, sc_ref[...], 0.0), axis=1, keepdims=True)
    out_ref[...] += y * w


def kernel(hidden_states, gate_w, gate_up_proj, down_proj):
    B, S, D = hidden_states.shape
    bs = B * S
    x = hidden_states.reshape(bs, D)
    out = pl.pallas_call(
        _moe_body,
        grid=(bs // TM, NE),
        in_specs=[
            pl.BlockSpec((TM, DM), lambda t, e: (t, 0)),
            pl.BlockSpec((DM, NE), lambda t, e: (0, 0)),
            pl.BlockSpec((1, DM, 2 * DF), lambda t, e: (e, 0, 0)),
            pl.BlockSpec((1, DF, DM), lambda t, e: (e, 0, 0)),
        ],
        out_specs=pl.BlockSpec((TM, DM), lambda t, e: (t, 0)),
        out_shape=jax.ShapeDtypeStruct((bs, DM), jnp.float32),
        scratch_shapes=[pltpu.VMEM((TM, NE), jnp.float32)],
    )(x, gate_w, gate_up_proj, down_proj)
    return out.reshape(B, S, D)


# speedup vs baseline: 1.0534x; 1.0534x over previous
import jax
import jax.numpy as jnp
from jax import lax
from jax.experimental import pallas as pl
from jax.experimental.pallas import tpu as pltpu

NE = 8
DM = 768
DF = 768
BS = 2048
TC = 1024


def _moe_body(x_ref, gw_ref, gu_ref, dn_ref, out_ref, sc_ref, xb_ref):
    e = pl.program_id(0)

    @pl.when(e == 0)
    def _router():
        x = x_ref[...]
        logits = jnp.dot(x, gw_ref[...], preferred_element_type=jnp.float32)
        eiota = lax.broadcasted_iota(jnp.int32, (BS, NE), 1)
        m1 = jnp.max(logits, axis=1, keepdims=True)
        a1 = jnp.min(jnp.where(logits == m1, eiota, NE), axis=1, keepdims=True)
        masked = jnp.where(eiota == a1, -jnp.inf, logits)
        m2 = jnp.max(masked, axis=1, keepdims=True)
        a2 = jnp.min(jnp.where(masked == m2, eiota, NE), axis=1, keepdims=True)
        w0 = 1.0 / (1.0 + jnp.exp(m2 - m1))
        w1 = 1.0 - w0
        sc_ref[...] = (jnp.where(eiota == a1, w0, 0.0)
                       + jnp.where(eiota == a2, w1, 0.0))
        out_ref[...] = jnp.zeros_like(out_ref)
        xb_ref[...] = x.astype(jnp.bfloat16)

    guw = gu_ref[0].astype(jnp.bfloat16)
    dnw = dn_ref[0].astype(jnp.bfloat16)
    eiota = lax.broadcasted_iota(jnp.int32, (TC, NE), 1)
    for ci in range(BS // TC):
        sl = pl.ds(ci * TC, TC)
        xb = xb_ref[sl, :]
        gu = jnp.dot(xb, guw, preferred_element_type=jnp.float32)
        g = gu[:, :DF]
        u = gu[:, DF:]
        h = (u * (g * jax.nn.sigmoid(g))).astype(jnp.bfloat16)
        y = jnp.dot(h, dnw, preferred_element_type=jnp.float32)
        w = jnp.sum(jnp.where(eiota == e, sc_ref[sl, :], 0.0),
                    axis=1, keepdims=True)
        out_ref[sl, :] += y * w


def kernel(hidden_states, gate_w, gate_up_proj, down_proj):
    B, S, D = hidden_states.shape
    x = hidden_states.reshape(B * S, D)
    out = pl.pallas_call(
        _moe_body,
        grid=(NE,),
        in_specs=[
            pl.BlockSpec((BS, DM), lambda e: (0, 0)),
            pl.BlockSpec((DM, NE), lambda e: (0, 0)),
            pl.BlockSpec((1, DM, 2 * DF), lambda e: (e, 0, 0)),
            pl.BlockSpec((1, DF, DM), lambda e: (e, 0, 0)),
        ],
        out_specs=pl.BlockSpec((BS, DM), lambda e: (0, 0)),
        out_shape=jax.ShapeDtypeStruct((BS, DM), jnp.float32),
        scratch_shapes=[
            pltpu.VMEM((BS, NE), jnp.float32),
            pltpu.VMEM((BS, DM), jnp.bfloat16),
        ],
    )(x, gate_w, gate_up_proj, down_proj)
    return out.reshape(B, S, D)
